# Initial kernel scaffold; baseline (speedup 1.0000x reference)
import jax
import jax.numpy as jnp
from jax.experimental import pallas as pl


def kernel(voxel_features, coors, nz_embed, W1, b1, g1, be1, W2, b2, g2, be2, batch_size):
    raise NotImplementedError("write your pallas kernel here")


# speedup vs baseline: 12.5493x; 12.5493x over previous
import functools

import jax
import jax.numpy as jnp
from jax import lax
from jax.experimental import pallas as pl
from jax.experimental.pallas import tpu as pltpu
from jax.experimental.pallas import tpu_sc as plsc

_NX, _NY, _NZ, _C, _B, _N = 400, 400, 4, 64, 2, 30000
_NW = 32
_SLICES = (_B * _NX) // _NW
_TILE_W = _SLICES * _NY
_CH = 2000
_NCHUNK = _N // _CH
_VREGS = _CH // 16


def _proj_body(w1_ref, f_ref, o_ref):
    o_ref[...] = lax.dot_general(
        w1_ref[...], f_ref[...], (((0,), (1,)), ((), ())),
        preferred_element_type=jnp.float32)


def _project(voxel_features, W1):
    return pl.pallas_call(
        _proj_body,
        out_shape=jax.ShapeDtypeStruct((_NZ, _N), jnp.float32),
    )(W1, voxel_features)


def _sc_body(cbh, czh, cxh, cyh, p0h, p1h, p2h, p3h, h1_hbm, bin_hbm,
             widx, h1acc, cb0, cb1, cb2, cb3, pb0, pb1, pb2, pb3):
    wid = lax.axis_index("s") * 2 + lax.axis_index("c")
    bxlo = wid * _SLICES
    lane = jnp.arange(16, dtype=jnp.int32)

    def _zero(i, carry):
        z16 = jnp.zeros((16,), jnp.float32)
        o = pl.multiple_of(i * 16, 8)
        widx[pl.ds(o, 16)] = z16
        h1acc[pl.ds(o, 16)] = z16
        return carry
    lax.fori_loop(0, (_NZ * _TILE_W) // 16, _zero, 0)

    def _keys(j):
        o = pl.multiple_of(j * 16, 8)
        b = jnp.minimum(cb0[pl.ds(o, 16)], _B - 1)
        z = jnp.minimum(cb1[pl.ds(o, 16)], _NZ - 1)
        x = jnp.minimum(cb2[pl.ds(o, 16)], _NX - 1)
        y = jnp.minimum(cb3[pl.ds(o, 16)], _NY - 1)
        bx = b * _NX + x
        msk = (bx >= bxlo) & (bx < bxlo + _SLICES)
        xl = jnp.clip(bx - bxlo, 0, _SLICES - 1)
        return z, xl, y, msk

    def _load_coors(base):
        pltpu.sync_copy(cbh.at[pl.ds(base, _CH)], cb0)
        pltpu.sync_copy(czh.at[pl.ds(base, _CH)], cb1)
        pltpu.sync_copy(cxh.at[pl.ds(base, _CH)], cb2)
        pltpu.sync_copy(cyh.at[pl.ds(base, _CH)], cb3)

    def _p1_chunk(ci, carry):
        base = pl.multiple_of(ci * _CH, 8)
        _load_coors(base)

        def _p1_vec(j, c2):
            z, xl, y, msk = _keys(j)
            idx = z * _TILE_W + xl * _NY + y
            pid = (lane + (base + j * 16 + 1)).astype(jnp.float32)
            plsc.store_scatter(widx, [idx], pid, mask=msk)
            return c2
        return lax.fori_loop(0, _VREGS, _p1_vec, carry)
    lax.fori_loop(0, _NCHUNK, _p1_chunk, 0)

    def _p2_chunk(ci, carry):
        base = pl.multiple_of(ci * _CH, 8)
        _load_coors(base)
        pltpu.sync_copy(p0h.at[pl.ds(base, _CH)], pb0)
        pltpu.sync_copy(p1h.at[pl.ds(base, _CH)], pb1)
        pltpu.sync_copy(p2h.at[pl.ds(base, _CH)], pb2)
        pltpu.sync_copy(p3h.at[pl.ds(base, _CH)], pb3)

        def _p2_vec(j, c2):
            z, xl, y, msk = _keys(j)
            idx = z * _TILE_W + xl * _NY + y
            pid = (lane + (base + j * 16 + 1)).astype(jnp.float32)
            w = plsc.load_gather(widx, [idx], mask=msk)
            keep = msk & (w == pid)
            hidx = xl * _NY + y
            o = pl.multiple_of(j * 16, 8)
            plsc.addupdate_scatter(h1acc, [hidx], pb0[pl.ds(o, 16)], mask=keep)
            plsc.addupdate_scatter(h1acc, [hidx + _TILE_W],
                                   pb1[pl.ds(o, 16)], mask=keep)
            plsc.addupdate_scatter(h1acc, [hidx + 2 * _TILE_W],
                                   pb2[pl.ds(o, 16)], mask=keep)
            plsc.addupdate_scatter(h1acc, [hidx + 3 * _TILE_W],
                                   pb3[pl.ds(o, 16)], mask=keep)
            return c2
        return lax.fori_loop(0, _VREGS, _p2_vec, carry)
    lax.fori_loop(0, _NCHUNK, _p2_chunk, 0)

    def _conv(i, carry):
        o = pl.multiple_of(i * 16, 8)
        w = widx[pl.ds(o, 16)]
        widx[pl.ds(o, 16)] = jnp.where(w > 0.0, 1.0, 0.0)
        return carry
    lax.fori_loop(0, (_NZ * _TILE_W) // 16, _conv, 0)

    bb = wid // (_NX // _SLICES)
    xlo = bxlo - bb * _NX
    for z in range(_NZ):
        off = pl.multiple_of(((bb * _NZ + z) * _NX + xlo) * _NY, 8)
        pltpu.sync_copy(widx.at[pl.ds(z * _TILE_W, _TILE_W)],
                        bin_hbm.at[pl.ds(off, _TILE_W)])
    for c in range(_NZ):
        off = pl.multiple_of(c * (_B * _NX * _NY) + wid * _TILE_W, 8)
        pltpu.sync_copy(h1acc.at[pl.ds(c * _TILE_W, _TILE_W)],
                        h1_hbm.at[pl.ds(off, _TILE_W)])


def _sc_scatter(cb, cz, cx, cy, p0, p1, p2, p3):
    mesh = plsc.VectorSubcoreMesh(core_axis_name="c", subcore_axis_name="s")
    fn = functools.partial(
        pl.kernel, mesh=mesh,
        compiler_params=pltpu.CompilerParams(needs_layout_passes=False),
        out_type=[
            jax.ShapeDtypeStruct((_NZ * _B * _NX * _NY,), jnp.float32),
            jax.ShapeDtypeStruct((_B * _NZ * _NX * _NY,), jnp.float32),
        ],
        scratch_types=[
            pltpu.VMEM((_NZ * _TILE_W,), jnp.float32),
            pltpu.VMEM((_NZ * _TILE_W,), jnp.float32),
            pltpu.VMEM((_CH,), jnp.int32),
            pltpu.VMEM((_CH,), jnp.int32),
            pltpu.VMEM((_CH,), jnp.int32),
            pltpu.VMEM((_CH,), jnp.int32),
            pltpu.VMEM((_CH,), jnp.float32),
            pltpu.VMEM((_CH,), jnp.float32),
            pltpu.VMEM((_CH,), jnp.float32),
            pltpu.VMEM((_CH,), jnp.float32),
        ],
    )(_sc_body)
    return fn(cb, cz, cx, cy, p0, p1, p2, p3)


def _head_body(p_ref, h1_ref, o_ref):
    a = [h1_ref[c, 0] + p_ref[0, c] for c in range(4)]
    m = (a[0] + a[1] + a[2] + a[3]) * 0.25
    d = [a[c] - m for c in range(4)]
    var = (d[0] * d[0] + d[1] * d[1] + d[2] * d[2] + d[3] * d[3]) * 0.25
    inv = lax.rsqrt(var + 1e-5)
    r = [jnp.maximum(d[c] * inv * p_ref[1, c] + p_ref[2, c], 0.0)
         for c in range(4)]
    s = [r[0] * p_ref[6, j] + r[1] * p_ref[7, j] + r[2] * p_ref[8, j]
         + r[3] * p_ref[9, j] + p_ref[3, j] for j in range(4)]
    m2 = (s[0] + s[1] + s[2] + s[3]) * 0.25
    d2 = [s[j] - m2 for j in range(4)]
    var2 = (d2[0] * d2[0] + d2[1] * d2[1] + d2[2] * d2[2]
            + d2[3] * d2[3]) * 0.25
    inv2 = lax.rsqrt(var2 + 1e-5)
    for j in range(4):
        o_ref[0, j] = d2[j] * inv2 * p_ref[4, j] + p_ref[5, j]


def _head(P, H1):
    bx = 40
    return pl.pallas_call(
        _head_body,
        grid=(_B, _NX // bx),
        in_specs=[
            pl.BlockSpec(memory_space=pltpu.SMEM),
            pl.BlockSpec((_NZ, 1, bx, _NY), lambda b, i: (0, b, i, 0)),
        ],
        out_specs=pl.BlockSpec((1, _NZ, bx, _NY), lambda b, i: (b, 0, i, 0)),
        out_shape=jax.ShapeDtypeStruct((_B, _NZ, _NX, _NY), jnp.float32),
    )(P, H1)


def kernel(voxel_features, coors, nz_embed, W1, b1, g1, be1, W2, b2, g2, be2,
           batch_size):
    projT = _project(voxel_features, W1)
    h1_flat, bin_flat = _sc_scatter(
        coors[:, 0], coors[:, 1], coors[:, 2], coors[:, 3],
        projT[0], projT[1], projT[2], projT[3])
    cvec = jnp.sum(nz_embed) * jnp.sum(W1, axis=0) + b1
    P = jnp.stack([cvec, g1, be1, b2, g2, be2, W2[0], W2[1], W2[2], W2[3]])
    H1 = h1_flat.reshape(_NZ, _B, _NX, _NY)
    out = _head(P, H1)
    binary = bin_flat.reshape(_B, _NZ, _NX, _NY)
    return out, binary


# speedup vs baseline: 18.7011x; 1.4902x over previous
import functools

import jax
import jax.numpy as jnp
from jax import lax
from jax.experimental import pallas as pl
from jax.experimental.pallas import tpu as pltpu
from jax.experimental.pallas import tpu_sc as plsc

_NX, _NY, _NZ, _C, _B, _N = 400, 400, 4, 64, 2, 30000
_NW = 32
_SLICES = (_B * _NX) // _NW
_TILE_W = _SLICES * _NY
_CW = _NZ * _TILE_W
_CH = 6000
_NCHUNK = _N // _CH
_VREGS = _CH // 16
_UNROLL = 5


def _proj_body(w1_ref, f_ref, ct_ref, p_ref, gk_ref, bx_ref):
    p_ref[...] = lax.dot_general(
        w1_ref[...], f_ref[...], (((0,), (1,)), ((), ())),
        preferred_element_type=jnp.float32)
    b = jnp.minimum(ct_ref[0:1, :], _B - 1)
    z = jnp.minimum(ct_ref[1:2, :], _NZ - 1)
    x = jnp.minimum(ct_ref[2:3, :], _NX - 1)
    y = jnp.minimum(ct_ref[3:4, :], _NY - 1)
    bx = b * _NX + x
    bx_ref[...] = bx
    gk_ref[...] = z * _TILE_W + bx * _NY + y


def _project(voxel_features, W1, coorsT):
    return pl.pallas_call(
        _proj_body,
        out_shape=[
            jax.ShapeDtypeStruct((_NZ, _N), jnp.float32),
            jax.ShapeDtypeStruct((1, _N), jnp.int32),
            jax.ShapeDtypeStruct((1, _N), jnp.int32),
        ],
    )(W1, voxel_features, coorsT)


def _sc_body(gkh, bxh, p0h, p1h, p2h, p3h, h1_hbm, bin_hbm,
             widx, h1acc, bgk, bbx, bp0, bp1, bp2, bp3):
    wid = lax.axis_index("s") * 2 + lax.axis_index("c")
    bxlo = wid * _SLICES
    lane = jnp.arange(16, dtype=jnp.int32)

    def _zero(i, carry):
        z16 = jnp.zeros((16,), jnp.float32)
        o = pl.multiple_of(i * 16, 8)
        widx[pl.ds(o, 16)] = z16
        h1acc[pl.ds(o, 16)] = z16
        return carry
    lax.fori_loop(0, _CW // 16, _zero, 0)

    def _keys(j):
        o = pl.multiple_of(j * 16, 8)
        bx = bbx[pl.ds(o, 16)]
        msk = (bx >= bxlo) & (bx < bxlo + _SLICES)
        idx = bgk[pl.ds(o, 16)] - bxlo * _NY
        idx = jnp.clip(idx, 0, _CW - 1)
        return o, idx, msk

    def _p1_chunk(ci, carry):
        base = pl.multiple_of(ci * _CH, 8)
        pltpu.sync_copy(gkh.at[pl.ds(base, _CH)], bgk)
        pltpu.sync_copy(bxh.at[pl.ds(base, _CH)], bbx)

        def _p1_vec(jj, c2):
            for u in range(_UNROLL):
                j = jj * _UNROLL + u
                o, idx, msk = _keys(j)
                pid = (lane + (base + j * 16 + 1)).astype(jnp.float32)
                plsc.store_scatter(widx, [idx], pid, mask=msk)
            return c2
        return lax.fori_loop(0, _VREGS // _UNROLL, _p1_vec, carry)
    lax.fori_loop(0, _NCHUNK, _p1_chunk, 0)

    def _p2_chunk(ci, carry):
        base = pl.multiple_of(ci * _CH, 8)
        pltpu.sync_copy(gkh.at[pl.ds(base, _CH)], bgk)
        pltpu.sync_copy(bxh.at[pl.ds(base, _CH)], bbx)
        pltpu.sync_copy(p0h.at[pl.ds(base, _CH)], bp0)
        pltpu.sync_copy(p1h.at[pl.ds(base, _CH)], bp1)
        pltpu.sync_copy(p2h.at[pl.ds(base, _CH)], bp2)
        pltpu.sync_copy(p3h.at[pl.ds(base, _CH)], bp3)

        def _p2_vec(jj, c2):
            for u in range(_UNROLL):
                j = jj * _UNROLL + u
                o, idx, msk = _keys(j)
                pid = (lane + (base + j * 16 + 1)).astype(jnp.float32)
                w = plsc.load_gather(widx, [idx], mask=msk)
                keep = msk & (w == pid)
                hidx = idx - jnp.where(idx >= 2 * _TILE_W, 2 * _TILE_W, 0)
                hidx = hidx - jnp.where(hidx >= _TILE_W, _TILE_W, 0)
                plsc.addupdate_scatter(h1acc, [hidx], bp0[pl.ds(o, 16)],
                                       mask=keep)
                plsc.addupdate_scatter(h1acc, [hidx + _TILE_W],
                                       bp1[pl.ds(o, 16)], mask=keep)
                plsc.addupdate_scatter(h1acc, [hidx + 2 * _TILE_W],
                                       bp2[pl.ds(o, 16)], mask=keep)
                plsc.addupdate_scatter(h1acc, [hidx + 3 * _TILE_W],
                                       bp3[pl.ds(o, 16)], mask=keep)
            return c2
        return lax.fori_loop(0, _VREGS // _UNROLL, _p2_vec, carry)
    lax.fori_loop(0, _NCHUNK, _p2_chunk, 0)

    def _conv(i, carry):
        o = pl.multiple_of(i * 16, 8)
        w = widx[pl.ds(o, 16)]
        widx[pl.ds(o, 16)] = jnp.where(w > 0.0, 1.0, 0.0)
        return carry
    lax.fori_loop(0, _CW // 16, _conv, 0)

    bb = wid // (_NX // _SLICES)
    xlo = bxlo - bb * _NX
    for z in range(_NZ):
        off = pl.multiple_of(((bb * _NZ + z) * _NX + xlo) * _NY, 8)
        pltpu.sync_copy(widx.at[pl.ds(z * _TILE_W, _TILE_W)],
                        bin_hbm.at[pl.ds(off, _TILE_W)])
    for c in range(_NZ):
        off = pl.multiple_of(c * (_B * _NX * _NY) + wid * _TILE_W, 8)
        pltpu.sync_copy(h1acc.at[pl.ds(c * _TILE_W, _TILE_W)],
                        h1_hbm.at[pl.ds(off, _TILE_W)])


def _sc_scatter(gk, bx, p0, p1, p2, p3):
    mesh = plsc.VectorSubcoreMesh(core_axis_name="c", subcore_axis_name="s")
    fn = functools.partial(
        pl.kernel, mesh=mesh,
        compiler_params=pltpu.CompilerParams(needs_layout_passes=False),
        out_type=[
            jax.ShapeDtypeStruct((_NZ * _B * _NX * _NY,), jnp.float32),
            jax.ShapeDtypeStruct((_B * _NZ * _NX * _NY,), jnp.float32),
        ],
        scratch_types=[
            pltpu.VMEM((_CW,), jnp.float32),
            pltpu.VMEM((_CW,), jnp.float32),
            pltpu.VMEM((_CH,), jnp.int32),
            pltpu.VMEM((_CH,), jnp.int32),
            pltpu.VMEM((_CH,), jnp.float32),
            pltpu.VMEM((_CH,), jnp.float32),
            pltpu.VMEM((_CH,), jnp.float32),
            pltpu.VMEM((_CH,), jnp.float32),
        ],
    )(_sc_body)
    return fn(gk, bx, p0, p1, p2, p3)


def _head_body(p_ref, h1_ref, o_ref):
    a = [h1_ref[c, 0] + p_ref[0, c] for c in range(4)]
    m = (a[0] + a[1] + a[2] + a[3]) * 0.25
    d = [a[c] - m for c in range(4)]
    var = (d[0] * d[0] + d[1] * d[1] + d[2] * d[2] + d[3] * d[3]) * 0.25
    inv = lax.rsqrt(var + 1e-5)
    r = [jnp.maximum(d[c] * inv * p_ref[1, c] + p_ref[2, c], 0.0)
         for c in range(4)]
    s = [r[0] * p_ref[6, j] + r[1] * p_ref[7, j] + r[2] * p_ref[8, j]
         + r[3] * p_ref[9, j] + p_ref[3, j] for j in range(4)]
    m2 = (s[0] + s[1] + s[2] + s[3]) * 0.25
    d2 = [s[j] - m2 for j in range(4)]
    var2 = (d2[0] * d2[0] + d2[1] * d2[1] + d2[2] * d2[2]
            + d2[3] * d2[3]) * 0.25
    inv2 = lax.rsqrt(var2 + 1e-5)
    for j in range(4):
        o_ref[0, j] = d2[j] * inv2 * p_ref[4, j] + p_ref[5, j]


def _head(P, H1):
    bx = 40
    return pl.pallas_call(
        _head_body,
        grid=(_B, _NX // bx),
        in_specs=[
            pl.BlockSpec(memory_space=pltpu.SMEM),
            pl.BlockSpec((_NZ, 1, bx, _NY), lambda b, i: (0, b, i, 0)),
        ],
        out_specs=pl.BlockSpec((1, _NZ, bx, _NY), lambda b, i: (b, 0, i, 0)),
        out_shape=jax.ShapeDtypeStruct((_B, _NZ, _NX, _NY), jnp.float32),
    )(P, H1)


def kernel(voxel_features, coors, nz_embed, W1, b1, g1, be1, W2, b2, g2, be2,
           batch_size):
    projT, gk, bx = _project(voxel_features, W1, coors.T)
    h1_flat, bin_flat = _sc_scatter(
        gk.reshape(_N), bx.reshape(_N),
        projT[0], projT[1], projT[2], projT[3])
    cvec = jnp.sum(nz_embed) * jnp.sum(W1, axis=0) + b1
    P = jnp.stack([cvec, g1, be1, b2, g2, be2, W2[0], W2[1], W2[2], W2[3]])
    H1 = h1_flat.reshape(_NZ, _B, _NX, _NY)
    out = _head(P, H1)
    binary = bin_flat.reshape(_B, _NZ, _NX, _NY)
    return out, binary


# speedup vs baseline: 22.4450x; 1.2002x over previous
import functools

import jax
import jax.numpy as jnp
from jax import lax
from jax.experimental import pallas as pl
from jax.experimental.pallas import tpu as pltpu
from jax.experimental.pallas import tpu_sc as plsc

_NX, _NY, _NZ, _C, _B, _N = 400, 400, 4, 64, 2, 30000
_NW = 32
_SLICES = (_B * _NX) // _NW
_TILE_W = _SLICES * _NY
_CW = _NZ * _TILE_W
_CH = 2000
_NCHUNK = _N // _CH
_VREGS = _CH // 16
_UNROLL = 5


def _proj_body(w1_ref, f_ref, ct_ref, p0_ref, p1_ref, p2_ref, p3_ref,
               gk_ref, bx_ref):
    p = lax.dot_general(
        w1_ref[...], f_ref[...], (((0,), (1,)), ((), ())),
        preferred_element_type=jnp.float32)
    p0_ref[...] = p[0:1, :]
    p1_ref[...] = p[1:2, :]
    p2_ref[...] = p[2:3, :]
    p3_ref[...] = p[3:4, :]
    b = jnp.minimum(ct_ref[0:1, :], _B - 1)
    z = jnp.minimum(ct_ref[1:2, :], _NZ - 1)
    x = jnp.minimum(ct_ref[2:3, :], _NX - 1)
    y = jnp.minimum(ct_ref[3:4, :], _NY - 1)
    bx = b * _NX + x
    bx_ref[...] = bx
    gk_ref[...] = z * _TILE_W + bx * _NY + y


def _project(voxel_features, W1, coorsT):
    row_f = jax.ShapeDtypeStruct((1, _N), jnp.float32)
    row_i = jax.ShapeDtypeStruct((1, _N), jnp.int32)
    return pl.pallas_call(
        _proj_body,
        out_shape=[row_f, row_f, row_f, row_f, row_i, row_i],
    )(W1, voxel_features, coorsT)


def _sc_body(gkh, bxh, p0h, p1h, p2h, p3h, zh, h1_hbm, bin_hbm,
             widx, h1acc, bgk0, bgk1, bbx0, bbx1, bp00, bp01, bp10, bp11,
             bp20, bp21, bp30, bp31, sem):
    bgk = [bgk0, bgk1]
    bbx = [bbx0, bbx1]
    bp0 = [bp00, bp01]
    bp1 = [bp10, bp11]
    bp2 = [bp20, bp21]
    bp3 = [bp30, bp31]
    wid = lax.axis_index("s") * 2 + lax.axis_index("c")
    bxlo = wid * _SLICES
    lane = jnp.arange(16, dtype=jnp.int32)

    pltpu.sync_copy(zh, widx)
    pltpu.sync_copy(zh, h1acc)

    def _keys(bgk_s, bbx_s, j):
        o = pl.multiple_of(j * 16, 8)
        bx = bbx_s[pl.ds(o, 16)]
        msk = (bx >= bxlo) & (bx < bxlo + _SLICES)
        idx = bgk_s[pl.ds(o, 16)] - bxlo * _NY
        idx = jnp.clip(idx, 0, _CW - 1)
        return o, idx, msk

    def _issue1(ci, s):
        base = pl.multiple_of(ci * _CH, 8)
        return [pltpu.async_copy(gkh.at[pl.ds(base, _CH)], bgk[s], sem),
                pltpu.async_copy(bxh.at[pl.ds(base, _CH)], bbx[s], sem)]

    hs = _issue1(0, 0)
    for ci in range(_NCHUNK):
        s = ci % 2
        for h in hs:
            h.wait()
        if ci + 1 < _NCHUNK:
            hs = _issue1(ci + 1, 1 - s)
        base = ci * _CH

        def _p1_vec(jj, c2, s=s, base=base):
            for u in range(_UNROLL):
                j = jj * _UNROLL + u
                o, idx, msk = _keys(bgk[s], bbx[s], j)
                pid = (lane + (base + j * 16 + 1)).astype(jnp.float32)
                plsc.store_scatter(widx, [idx], pid, mask=msk)
            return c2
        lax.fori_loop(0, _VREGS // _UNROLL, _p1_vec, 0)

    def _issue2(ci, s):
        base = pl.multiple_of(ci * _CH, 8)
        return [pltpu.async_copy(gkh.at[pl.ds(base, _CH)], bgk[s], sem),
                pltpu.async_copy(bxh.at[pl.ds(base, _CH)], bbx[s], sem),
                pltpu.async_copy(p0h.at[pl.ds(base, _CH)], bp0[s], sem),
                pltpu.async_copy(p1h.at[pl.ds(base, _CH)], bp1[s], sem),
                pltpu.async_copy(p2h.at[pl.ds(base, _CH)], bp2[s], sem),
                pltpu.async_copy(p3h.at[pl.ds(base, _CH)], bp3[s], sem)]

    hs = _issue2(0, 0)
    for ci in range(_NCHUNK):
        s = ci % 2
        for h in hs:
            h.wait()
        if ci + 1 < _NCHUNK:
            hs = _issue2(ci + 1, 1 - s)
        base = ci * _CH

        def _p2_vec(jj, c2, s=s, base=base):
            for u in range(_UNROLL):
                j = jj * _UNROLL + u
                o, idx, msk = _keys(bgk[s], bbx[s], j)
                pid = (lane + (base + j * 16 + 1)).astype(jnp.float32)
                w = plsc.load_gather(widx, [idx], mask=msk)
                keep = msk & (w == pid)
                hidx = idx - jnp.where(idx >= 2 * _TILE_W, 2 * _TILE_W, 0)
                hidx = hidx - jnp.where(hidx >= _TILE_W, _TILE_W, 0)
                plsc.addupdate_scatter(h1acc, [hidx], bp0[s][pl.ds(o, 16)],
                                       mask=keep)
                plsc.addupdate_scatter(h1acc, [hidx + _TILE_W],
                                       bp1[s][pl.ds(o, 16)], mask=keep)
                plsc.addupdate_scatter(h1acc, [hidx + 2 * _TILE_W],
                                       bp2[s][pl.ds(o, 16)], mask=keep)
                plsc.addupdate_scatter(h1acc, [hidx + 3 * _TILE_W],
                                       bp3[s][pl.ds(o, 16)], mask=keep)
            return c2
        lax.fori_loop(0, _VREGS // _UNROLL, _p2_vec, 0)

    def _conv(i, carry):
        for u in range(10):
            o = pl.multiple_of((i * 10 + u) * 16, 8)
            w = widx[pl.ds(o, 16)]
            widx[pl.ds(o, 16)] = jnp.where(w > 0.0, 1.0, 0.0)
        return carry
    lax.fori_loop(0, _CW // 160, _conv, 0)

    bb = wid // (_NX // _SLICES)
    xlo = bxlo - bb * _NX
    for z in range(_NZ):
        off = pl.multiple_of(((bb * _NZ + z) * _NX + xlo) * _NY, 8)
        pltpu.sync_copy(widx.at[pl.ds(z * _TILE_W, _TILE_W)],
                        bin_hbm.at[pl.ds(off, _TILE_W)])
    for c in range(_NZ):
        off = pl.multiple_of(c * (_B * _NX * _NY) + wid * _TILE_W, 8)
        pltpu.sync_copy(h1acc.at[pl.ds(c * _TILE_W, _TILE_W)],
                        h1_hbm.at[pl.ds(off, _TILE_W)])


def _sc_scatter(gk, bx, p0, p1, p2, p3, zeros_cw):
    mesh = plsc.VectorSubcoreMesh(core_axis_name="c", subcore_axis_name="s")
    fn = functools.partial(
        pl.kernel, mesh=mesh,
        compiler_params=pltpu.CompilerParams(needs_layout_passes=False),
        out_type=[
            jax.ShapeDtypeStruct((_NZ * _B * _NX * _NY,), jnp.float32),
            jax.ShapeDtypeStruct((_B * _NZ * _NX * _NY,), jnp.float32),
        ],
        scratch_types=[
            pltpu.VMEM((_CW,), jnp.float32),
            pltpu.VMEM((_CW,), jnp.float32),
            pltpu.VMEM((_CH,), jnp.int32),
            pltpu.VMEM((_CH,), jnp.int32),
            pltpu.VMEM((_CH,), jnp.int32),
            pltpu.VMEM((_CH,), jnp.int32),
            pltpu.VMEM((_CH,), jnp.float32),
            pltpu.VMEM((_CH,), jnp.float32),
            pltpu.VMEM((_CH,), jnp.float32),
            pltpu.VMEM((_CH,), jnp.float32),
            pltpu.VMEM((_CH,), jnp.float32),
            pltpu.VMEM((_CH,), jnp.float32),
            pltpu.VMEM((_CH,), jnp.float32),
            pltpu.VMEM((_CH,), jnp.float32),
            pltpu.SemaphoreType.DMA,
        ],
    )(_sc_body)
    return fn(gk, bx, p0, p1, p2, p3, zeros_cw)


def _head_body(p_ref, h1_ref, o_ref):
    a = [h1_ref[c, 0] + p_ref[0, c] for c in range(4)]
    m = (a[0] + a[1] + a[2] + a[3]) * 0.25
    d = [a[c] - m for c in range(4)]
    var = (d[0] * d[0] + d[1] * d[1] + d[2] * d[2] + d[3] * d[3]) * 0.25
    inv = lax.rsqrt(var + 1e-5)
    r = [jnp.maximum(d[c] * inv * p_ref[1, c] + p_ref[2, c], 0.0)
         for c in range(4)]
    s = [r[0] * p_ref[6, j] + r[1] * p_ref[7, j] + r[2] * p_ref[8, j]
         + r[3] * p_ref[9, j] + p_ref[3, j] for j in range(4)]
    m2 = (s[0] + s[1] + s[2] + s[3]) * 0.25
    d2 = [s[j] - m2 for j in range(4)]
    var2 = (d2[0] * d2[0] + d2[1] * d2[1] + d2[2] * d2[2]
            + d2[3] * d2[3]) * 0.25
    inv2 = lax.rsqrt(var2 + 1e-5)
    for j in range(4):
        o_ref[0, j] = d2[j] * inv2 * p_ref[4, j] + p_ref[5, j]


def _head(P, H1):
    bx = 40
    return pl.pallas_call(
        _head_body,
        grid=(_B, _NX // bx),
        in_specs=[
            pl.BlockSpec(memory_space=pltpu.SMEM),
            pl.BlockSpec((_NZ, 1, bx, _NY), lambda b, i: (0, b, i, 0)),
        ],
        out_specs=pl.BlockSpec((1, _NZ, bx, _NY), lambda b, i: (b, 0, i, 0)),
        out_shape=jax.ShapeDtypeStruct((_B, _NZ, _NX, _NY), jnp.float32),
    )(P, H1)


def kernel(voxel_features, coors, nz_embed, W1, b1, g1, be1, W2, b2, g2, be2,
           batch_size):
    p0, p1, p2, p3, gk, bx = _project(voxel_features, W1, coors.T)
    h1_flat, bin_flat = _sc_scatter(
        gk.reshape(_N), bx.reshape(_N),
        p0.reshape(_N), p1.reshape(_N), p2.reshape(_N), p3.reshape(_N),
        jnp.zeros((_CW,), jnp.float32))
    cvec = jnp.sum(nz_embed) * jnp.sum(W1, axis=0) + b1
    P = jnp.stack([cvec, g1, be1, b2, g2, be2, W2[0], W2[1], W2[2], W2[3]])
    H1 = h1_flat.reshape(_NZ, _B, _NX, _NY)
    out = _head(P, H1)
    binary = bin_flat.reshape(_B, _NZ, _NX, _NY)
    return out, binary


# speedup vs baseline: 24.0536x; 1.0717x over previous
import functools

import jax
import jax.numpy as jnp
from jax import lax
from jax.experimental import pallas as pl
from jax.experimental.pallas import tpu as pltpu
from jax.experimental.pallas import tpu_sc as plsc

_NX, _NY, _NZ, _C, _B, _N = 400, 400, 4, 64, 2, 30000
_NW = 32
_SLICES = (_B * _NX) // _NW
_TILE_W = _SLICES * _NY
_CW = _NZ * _TILE_W
_CH = 2000
_NCHUNK = _N // _CH
_VREGS = _CH // 16
_UNROLL = 5


def _proj_body(w1_ref, f_ref, ct_ref, ne_ref, b1_ref, g1_ref, be1_ref,
               w2_ref, b2_ref, g2_ref, be2_ref,
               p0_ref, p1_ref, p2_ref, p3_ref, gk_ref, bx_ref, prm_ref):
    p = lax.dot_general(
        w1_ref[...], f_ref[...], (((0,), (1,)), ((), ())),
        preferred_element_type=jnp.float32)
    p0_ref[...] = p[0, :]
    p1_ref[...] = p[1, :]
    p2_ref[...] = p[2, :]
    p3_ref[...] = p[3, :]
    b = jnp.minimum(ct_ref[0:1, :], _B - 1)
    z = jnp.minimum(ct_ref[1:2, :], _NZ - 1)
    x = jnp.minimum(ct_ref[2:3, :], _NX - 1)
    y = jnp.minimum(ct_ref[3:4, :], _NY - 1)
    bx = b * _NX + x
    bx_ref[...] = bx[0, :]
    gk_ref[...] = (z * _TILE_W + bx * _NY + y)[0, :]
    cvec = jnp.sum(ne_ref[...]) * jnp.sum(w1_ref[...], axis=0) + b1_ref[...]
    prm_ref[0:1, :] = cvec.reshape(1, _NZ)
    prm_ref[1:2, :] = g1_ref[...].reshape(1, _NZ)
    prm_ref[2:3, :] = be1_ref[...].reshape(1, _NZ)
    prm_ref[3:4, :] = b2_ref[...].reshape(1, _NZ)
    prm_ref[4:5, :] = g2_ref[...].reshape(1, _NZ)
    prm_ref[5:6, :] = be2_ref[...].reshape(1, _NZ)
    prm_ref[6:10, :] = w2_ref[...]


def _project(voxel_features, W1, coorsT, nz_embed, b1, g1, be1, W2, b2, g2,
             be2):
    vec_f = jax.ShapeDtypeStruct((_N,), jnp.float32)
    vec_i = jax.ShapeDtypeStruct((_N,), jnp.int32)
    return pl.pallas_call(
        _proj_body,
        out_shape=[vec_f, vec_f, vec_f, vec_f, vec_i, vec_i,
                   jax.ShapeDtypeStruct((10, _NZ), jnp.float32)],
    )(W1, voxel_features, coorsT, nz_embed, b1, g1, be1, W2, b2, g2, be2)


def _sc_body(gkh, bxh, p0h, p1h, p2h, p3h, zh, h1_hbm, bin_hbm,
             widx, h1acc, bgk0, bgk1, bbx0, bbx1, bp00, bp01, bp10, bp11,
             bp20, bp21, bp30, bp31, sem):
    bgk = [bgk0, bgk1]
    bbx = [bbx0, bbx1]
    bp0 = [bp00, bp01]
    bp1 = [bp10, bp11]
    bp2 = [bp20, bp21]
    bp3 = [bp30, bp31]
    wid = lax.axis_index("s") * 2 + lax.axis_index("c")
    bxlo = wid * _SLICES
    lane = jnp.arange(16, dtype=jnp.int32)

    pltpu.sync_copy(zh, widx)
    pltpu.sync_copy(zh, h1acc)

    def _keys(bgk_s, bbx_s, j):
        o = pl.multiple_of(j * 16, 8)
        bx = bbx_s[pl.ds(o, 16)]
        msk = (bx >= bxlo) & (bx < bxlo + _SLICES)
        idx = bgk_s[pl.ds(o, 16)] - bxlo * _NY
        idx = jnp.clip(idx, 0, _CW - 1)
        return o, idx, msk

    def _issue1(ci, s):
        base = pl.multiple_of(ci * _CH, 8)
        return [pltpu.async_copy(gkh.at[pl.ds(base, _CH)], bgk[s], sem),
                pltpu.async_copy(bxh.at[pl.ds(base, _CH)], bbx[s], sem)]

    hs = _issue1(0, 0)
    for ci in range(_NCHUNK):
        s = ci % 2
        for h in hs:
            h.wait()
        if ci + 1 < _NCHUNK:
            hs = _issue1(ci + 1, 1 - s)
        base = ci * _CH

        def _p1_vec(jj, c2, s=s, base=base):
            for u in range(_UNROLL):
                j = jj * _UNROLL + u
                o, idx, msk = _keys(bgk[s], bbx[s], j)
                pid = (lane + (base + j * 16 + 1)).astype(jnp.float32)
                plsc.store_scatter(widx, [idx], pid, mask=msk)
            return c2
        lax.fori_loop(0, _VREGS // _UNROLL, _p1_vec, 0)

    def _issue2(ci, s):
        base = pl.multiple_of(ci * _CH, 8)
        return [pltpu.async_copy(gkh.at[pl.ds(base, _CH)], bgk[s], sem),
                pltpu.async_copy(bxh.at[pl.ds(base, _CH)], bbx[s], sem),
                pltpu.async_copy(p0h.at[pl.ds(base, _CH)], bp0[s], sem),
                pltpu.async_copy(p1h.at[pl.ds(base, _CH)], bp1[s], sem),
                pltpu.async_copy(p2h.at[pl.ds(base, _CH)], bp2[s], sem),
                pltpu.async_copy(p3h.at[pl.ds(base, _CH)], bp3[s], sem)]

    hs = _issue2(0, 0)
    for ci in range(_NCHUNK):
        s = ci % 2
        for h in hs:
            h.wait()
        if ci + 1 < _NCHUNK:
            hs = _issue2(ci + 1, 1 - s)
        base = ci * _CH

        def _p2_vec(jj, c2, s=s, base=base):
            for u in range(_UNROLL):
                j = jj * _UNROLL + u
                o, idx, msk = _keys(bgk[s], bbx[s], j)
                pid = (lane + (base + j * 16 + 1)).astype(jnp.float32)
                w = plsc.load_gather(widx, [idx], mask=msk)
                keep = msk & (w == pid)
                hidx = idx - jnp.where(idx >= 2 * _TILE_W, 2 * _TILE_W, 0)
                hidx = hidx - jnp.where(hidx >= _TILE_W, _TILE_W, 0)
                plsc.addupdate_scatter(h1acc, [hidx], bp0[s][pl.ds(o, 16)],
                                       mask=keep)
                plsc.addupdate_scatter(h1acc, [hidx + _TILE_W],
                                       bp1[s][pl.ds(o, 16)], mask=keep)
                plsc.addupdate_scatter(h1acc, [hidx + 2 * _TILE_W],
                                       bp2[s][pl.ds(o, 16)], mask=keep)
                plsc.addupdate_scatter(h1acc, [hidx + 3 * _TILE_W],
                                       bp3[s][pl.ds(o, 16)], mask=keep)
            return c2
        lax.fori_loop(0, _VREGS // _UNROLL, _p2_vec, 0)

    def _conv(i, carry):
        for u in range(10):
            o = pl.multiple_of((i * 10 + u) * 16, 8)
            w = widx[pl.ds(o, 16)]
            widx[pl.ds(o, 16)] = jnp.where(w > 0.0, 1.0, 0.0)
        return carry
    lax.fori_loop(0, _CW // 160, _conv, 0)

    bb = wid // (_NX // _SLICES)
    xlo = bxlo - bb * _NX
    for z in range(_NZ):
        off = pl.multiple_of(((bb * _NZ + z) * _NX + xlo) * _NY, 8)
        pltpu.sync_copy(widx.at[pl.ds(z * _TILE_W, _TILE_W)],
                        bin_hbm.at[pl.ds(off, _TILE_W)])
    for c in range(_NZ):
        off = pl.multiple_of(c * (_B * _NX * _NY) + wid * _TILE_W, 8)
        pltpu.sync_copy(h1acc.at[pl.ds(c * _TILE_W, _TILE_W)],
                        h1_hbm.at[pl.ds(off, _TILE_W)])


def _sc_scatter(gk, bx, p0, p1, p2, p3, zeros_cw):
    mesh = plsc.VectorSubcoreMesh(core_axis_name="c", subcore_axis_name="s")
    fn = functools.partial(
        pl.kernel, mesh=mesh,
        compiler_params=pltpu.CompilerParams(needs_layout_passes=False),
        out_type=[
            jax.ShapeDtypeStruct((_NZ * _B * _NX * _NY,), jnp.float32),
            jax.ShapeDtypeStruct((_B * _NZ * _NX * _NY,), jnp.float32),
        ],
        scratch_types=[
            pltpu.VMEM((_CW,), jnp.float32),
            pltpu.VMEM((_CW,), jnp.float32),
            pltpu.VMEM((_CH,), jnp.int32),
            pltpu.VMEM((_CH,), jnp.int32),
            pltpu.VMEM((_CH,), jnp.int32),
            pltpu.VMEM((_CH,), jnp.int32),
            pltpu.VMEM((_CH,), jnp.float32),
            pltpu.VMEM((_CH,), jnp.float32),
            pltpu.VMEM((_CH,), jnp.float32),
            pltpu.VMEM((_CH,), jnp.float32),
            pltpu.VMEM((_CH,), jnp.float32),
            pltpu.VMEM((_CH,), jnp.float32),
            pltpu.VMEM((_CH,), jnp.float32),
            pltpu.VMEM((_CH,), jnp.float32),
            pltpu.SemaphoreType.DMA,
        ],
    )(_sc_body)
    return fn(gk, bx, p0, p1, p2, p3, zeros_cw)


def _head_body(p_ref, h1_ref, o_ref):
    a = [h1_ref[c, 0] + p_ref[0, c] for c in range(4)]
    m = (a[0] + a[1] + a[2] + a[3]) * 0.25
    d = [a[c] - m for c in range(4)]
    var = (d[0] * d[0] + d[1] * d[1] + d[2] * d[2] + d[3] * d[3]) * 0.25
    inv = lax.rsqrt(var + 1e-5)
    r = [jnp.maximum(d[c] * inv * p_ref[1, c] + p_ref[2, c], 0.0)
         for c in range(4)]
    s = [r[0] * p_ref[6, j] + r[1] * p_ref[7, j] + r[2] * p_ref[8, j]
         + r[3] * p_ref[9, j] + p_ref[3, j] for j in range(4)]
    m2 = (s[0] + s[1] + s[2] + s[3]) * 0.25
    d2 = [s[j] - m2 for j in range(4)]
    var2 = (d2[0] * d2[0] + d2[1] * d2[1] + d2[2] * d2[2]
            + d2[3] * d2[3]) * 0.25
    inv2 = lax.rsqrt(var2 + 1e-5)
    for j in range(4):
        o_ref[0, j] = d2[j] * inv2 * p_ref[4, j] + p_ref[5, j]


def _head(P, H1):
    bx = 40
    return pl.pallas_call(
        _head_body,
        grid=(_B, _NX // bx),
        in_specs=[
            pl.BlockSpec(memory_space=pltpu.SMEM),
            pl.BlockSpec((_NZ, 1, bx, _NY), lambda b, i: (0, b, i, 0)),
        ],
        out_specs=pl.BlockSpec((1, _NZ, bx, _NY), lambda b, i: (b, 0, i, 0)),
        out_shape=jax.ShapeDtypeStruct((_B, _NZ, _NX, _NY), jnp.float32),
    )(P, H1)


def kernel(voxel_features, coors, nz_embed, W1, b1, g1, be1, W2, b2, g2, be2,
           batch_size):
    p0, p1, p2, p3, gk, bx, P = _project(
        voxel_features, W1, coors.T, nz_embed, b1, g1, be1, W2, b2, g2, be2)
    h1_flat, bin_flat = _sc_scatter(
        gk, bx, p0, p1, p2, p3, jnp.zeros((_CW,), jnp.float32))
    H1 = h1_flat.reshape(_NZ, _B, _NX, _NY)
    out = _head(P, H1)
    binary = bin_flat.reshape(_B, _NZ, _NX, _NY)
    return out, binary
